# Initial kernel scaffold; baseline (speedup 1.0000x reference)
import jax
import jax.numpy as jnp
from jax.experimental import pallas as pl


def kernel(waveform):
    raise NotImplementedError("write your pallas kernel here")


# speedup vs baseline: 10.5541x; 10.5541x over previous
import numpy as np
import jax
import jax.numpy as jnp
from jax.experimental import pallas as pl
from jax.experimental.pallas import tpu as pltpu

_SR = 16000
_N_FFT = 512
_N_MELS = 80
_HOP = 160
_ALPHA, _DELTA, _R, _S, _EPS = 0.98, 2.0, 0.5, 0.025, 1e-6
_TC = 256
_NB = _N_FFT // 2


def _mel_fbanks_np(n_freqs, f_min, f_max, n_mels, sr):
    all_freqs = np.linspace(0.0, sr / 2.0, n_freqs)

    def hz_to_mel(f):
        return 2595.0 * np.log10(1.0 + f / 700.0)

    def mel_to_hz(m):
        return 700.0 * (10.0 ** (m / 2595.0) - 1.0)

    m_pts = np.linspace(hz_to_mel(f_min), hz_to_mel(f_max), n_mels + 2)
    f_pts = mel_to_hz(m_pts)
    f_diff = f_pts[1:] - f_pts[:-1]
    slopes = f_pts[None, :] - all_freqs[:, None]
    down = -slopes[:, :-2] / f_diff[:-1]
    up = slopes[:, 2:] / f_diff[1:]
    return np.clip(np.minimum(down, up), 0.0, None)


def _consts():
    n = np.arange(_N_FFT, dtype=np.float64)
    win = 0.5 * (1.0 - np.cos(2.0 * np.pi * n / _N_FFT))
    k = np.arange(1, _NB + 1, dtype=np.float64)
    ang = 2.0 * np.pi * np.outer(n, k) / _N_FFT
    dft = np.concatenate([win[:, None] * np.cos(ang),
                          win[:, None] * np.sin(ang)], axis=1)
    fbt = _mel_fbanks_np(_N_FFT // 2 + 1, 0.0, _SR / 2.0,
                         _N_MELS, _SR)[1:_NB + 1].T
    t = np.arange(_TC, dtype=np.float64)
    lt = np.where(t[None, :] >= t[:, None],
                  _S * (1.0 - _S) ** (t[None, :] - t[:, None]), 0.0)
    dt = ((1.0 - _S) ** (t + 1.0))[None, :]
    return (dft.astype(np.float32), fbt.astype(np.float32),
            lt.astype(np.float32), dt.astype(np.float32))


_DFT, _FBT, _LTM, _DTV = _consts()


def _body(x_ref, a_ref, fbt_ref, lt_ref, dt_ref, o_ref, s_ref):
    c = pl.program_id(1)
    base = pl.multiple_of(c * _TC, 8)
    hp = jax.lax.Precision.HIGHEST
    f32 = jnp.float32
    v = x_ref[0, pl.ds(base, _TC + 8), :]
    acc = jnp.dot(v[0:_TC], a_ref[0:160, :], precision=hp, preferred_element_type=f32)
    acc = acc + jnp.dot(v[1:_TC + 1], a_ref[160:320, :], precision=hp,
                        preferred_element_type=f32)
    acc = acc + jnp.dot(v[2:_TC + 2], a_ref[320:480, :], precision=hp,
                        preferred_element_type=f32)
    acc = acc + jnp.dot(v[3:_TC + 3, 0:32], a_ref[480:512, :], precision=hp,
                        preferred_element_type=f32)
    power = acc[:, :_NB] * acc[:, :_NB] + acc[:, _NB:] * acc[:, _NB:]
    x_t = jax.lax.dot_general(fbt_ref[...], power, (((1,), (1,)), ((), ())),
                              precision=hp, preferred_element_type=f32) + 1e-9

    @pl.when(c == 0)
    def _():
        s_ref[...] = x_t[:, 0:1]

    s_in = s_ref[...]
    smooth = (jnp.dot(x_t, lt_ref[...], precision=hp, preferred_element_type=f32)
              + s_in * dt_ref[...])
    s_ref[...] = smooth[:, _TC - 1:_TC]
    u = x_t * jnp.exp(-_ALPHA * jnp.log(smooth + _EPS)) + _DELTA
    o_ref[0] = jnp.sqrt(u) - _DELTA ** _R


def kernel(waveform):
    b, s = waveform.shape
    t_frames = 1 + s // _HOP
    nc = -(-t_frames // _TC)
    rows = nc * _TC + 8
    spad = rows * _HOP
    left = waveform[:, 256:0:-1]
    right = waveform[:, -2:-258:-1]
    z = jnp.zeros((b, spad - s - 2 * 256), waveform.dtype)
    xp = jnp.concatenate([left, waveform, right, z], axis=1).reshape(b, rows, _HOP)
    out = pl.pallas_call(
        _body,
        out_shape=jax.ShapeDtypeStruct((b, _N_MELS, t_frames), jnp.float32),
        grid=(b, nc),
        in_specs=[
            pl.BlockSpec((1, rows, _HOP), lambda bi, ci: (bi, 0, 0)),
            pl.BlockSpec((_N_FFT, 2 * _NB), lambda bi, ci: (0, 0)),
            pl.BlockSpec((_N_MELS, _NB), lambda bi, ci: (0, 0)),
            pl.BlockSpec((_TC, _TC), lambda bi, ci: (0, 0)),
            pl.BlockSpec((1, _TC), lambda bi, ci: (0, 0)),
        ],
        out_specs=pl.BlockSpec((1, _N_MELS, _TC), lambda bi, ci: (bi, 0, ci)),
        scratch_shapes=[pltpu.VMEM((_N_MELS, 1), jnp.float32)],
        compiler_params=pltpu.CompilerParams(
            dimension_semantics=("parallel", "arbitrary"),
            vmem_limit_bytes=48 * 1024 * 1024,
        ),
        name="mel_pcen_fused",
    )(xp, jnp.asarray(_DFT), jnp.asarray(_FBT), jnp.asarray(_LTM), jnp.asarray(_DTV))
    return out


# speedup vs baseline: 18.3350x; 1.7372x over previous
import numpy as np
import jax
import jax.numpy as jnp
from jax.experimental import pallas as pl
from jax.experimental.pallas import tpu as pltpu

_SR = 16000
_N_FFT = 512
_N_MELS = 80
_HOP = 160
_ALPHA, _DELTA, _R, _S, _EPS = 0.98, 2.0, 0.5, 0.025, 1e-6
_TC = 256
_NB = _N_FFT // 2


def _mel_fbanks_np(n_freqs, f_min, f_max, n_mels, sr):
    all_freqs = np.linspace(0.0, sr / 2.0, n_freqs)

    def hz_to_mel(f):
        return 2595.0 * np.log10(1.0 + f / 700.0)

    def mel_to_hz(m):
        return 700.0 * (10.0 ** (m / 2595.0) - 1.0)

    m_pts = np.linspace(hz_to_mel(f_min), hz_to_mel(f_max), n_mels + 2)
    f_pts = mel_to_hz(m_pts)
    f_diff = f_pts[1:] - f_pts[:-1]
    slopes = f_pts[None, :] - all_freqs[:, None]
    down = -slopes[:, :-2] / f_diff[:-1]
    up = slopes[:, 2:] / f_diff[1:]
    return np.clip(np.minimum(down, up), 0.0, None)


def _consts():
    n = np.arange(_N_FFT, dtype=np.float64)
    win = 0.5 * (1.0 - np.cos(2.0 * np.pi * n / _N_FFT))
    k = np.arange(1, _NB + 1, dtype=np.float64)
    ang = 2.0 * np.pi * np.outer(n, k) / _N_FFT
    dft = np.concatenate([win[:, None] * np.cos(ang),
                          win[:, None] * np.sin(ang)], axis=1)
    fbt = _mel_fbanks_np(_N_FFT // 2 + 1, 0.0, _SR / 2.0,
                         _N_MELS, _SR)[1:_NB + 1].T
    t = np.arange(_TC, dtype=np.float64)
    lt = np.where(t[None, :] >= t[:, None],
                  _S * (1.0 - _S) ** (t[None, :] - t[:, None]), 0.0)
    dt = ((1.0 - _S) ** (t + 1.0))[None, :]
    dft32 = dft.astype(np.float32)
    dft_hi = dft32.astype(jnp.bfloat16)
    dft_lo = (dft32 - dft_hi.astype(np.float32)).astype(jnp.bfloat16)
    return (dft_hi, dft_lo, fbt.astype(np.float32),
            lt.astype(np.float32), dt.astype(np.float32))


_DFT_HI, _DFT_LO, _FBT, _LTM, _DTV = _consts()


def _split_dot(x, a_hi_ref, a_lo_ref, lo, hi):
    bf16, f32 = jnp.bfloat16, jnp.float32
    xh = x.astype(bf16)
    xl = (x - xh.astype(f32)).astype(bf16)
    ah = a_hi_ref[lo:hi, :]
    al = a_lo_ref[lo:hi, :]
    out = jnp.dot(xh, ah, preferred_element_type=f32)
    out = out + (jnp.dot(xh, al, preferred_element_type=f32)
                 + jnp.dot(xl, ah, preferred_element_type=f32))
    return out


def _body(x_ref, ah_ref, al_ref, fbt_ref, lt_ref, dt_ref, o_ref, s_ref):
    c = pl.program_id(1)
    base = pl.multiple_of(c * _TC, 8)
    f32 = jnp.float32
    v = x_ref[0, pl.ds(base, _TC + 8), :]
    acc = _split_dot(v[0:_TC], ah_ref, al_ref, 0, 160)
    acc = acc + _split_dot(v[1:_TC + 1], ah_ref, al_ref, 160, 320)
    acc = acc + _split_dot(v[2:_TC + 2], ah_ref, al_ref, 320, 480)
    acc = acc + _split_dot(v[3:_TC + 3, 0:32], ah_ref, al_ref, 480, 512)
    power = acc[:, :_NB] * acc[:, :_NB] + acc[:, _NB:] * acc[:, _NB:]
    x_t = jax.lax.dot_general(fbt_ref[...], power, (((1,), (1,)), ((), ())),
                              preferred_element_type=f32) + 1e-9

    @pl.when(c == 0)
    def _():
        s_ref[...] = x_t[:, 0:1]

    s_in = s_ref[...]
    smooth = (jnp.dot(x_t, lt_ref[...], preferred_element_type=f32)
              + s_in * dt_ref[...])
    s_ref[...] = smooth[:, _TC - 1:_TC]
    u = x_t * jnp.exp(-_ALPHA * jnp.log(smooth + _EPS)) + _DELTA
    o_ref[0] = jnp.sqrt(u) - _DELTA ** _R


def kernel(waveform):
    b, s = waveform.shape
    t_frames = 1 + s // _HOP
    nc = -(-t_frames // _TC)
    rows = nc * _TC + 8
    spad = rows * _HOP
    left = waveform[:, 256:0:-1]
    right = waveform[:, -2:-258:-1]
    z = jnp.zeros((b, spad - s - 2 * 256), waveform.dtype)
    xp = jnp.concatenate([left, waveform, right, z], axis=1).reshape(b, rows, _HOP)
    out = pl.pallas_call(
        _body,
        out_shape=jax.ShapeDtypeStruct((b, _N_MELS, t_frames), jnp.float32),
        grid=(b, nc),
        in_specs=[
            pl.BlockSpec((1, rows, _HOP), lambda bi, ci: (bi, 0, 0)),
            pl.BlockSpec((_N_FFT, 2 * _NB), lambda bi, ci: (0, 0)),
            pl.BlockSpec((_N_FFT, 2 * _NB), lambda bi, ci: (0, 0)),
            pl.BlockSpec((_N_MELS, _NB), lambda bi, ci: (0, 0)),
            pl.BlockSpec((_TC, _TC), lambda bi, ci: (0, 0)),
            pl.BlockSpec((1, _TC), lambda bi, ci: (0, 0)),
        ],
        out_specs=pl.BlockSpec((1, _N_MELS, _TC), lambda bi, ci: (bi, 0, ci)),
        scratch_shapes=[pltpu.VMEM((_N_MELS, 1), jnp.float32)],
        compiler_params=pltpu.CompilerParams(
            dimension_semantics=("parallel", "arbitrary"),
            vmem_limit_bytes=48 * 1024 * 1024,
        ),
        name="mel_pcen_fused",
    )(xp, jnp.asarray(_DFT_HI), jnp.asarray(_DFT_LO), jnp.asarray(_FBT),
      jnp.asarray(_LTM), jnp.asarray(_DTV))
    return out


# speedup vs baseline: 23.4435x; 1.2786x over previous
import numpy as np
import jax
import jax.numpy as jnp
from jax.experimental import pallas as pl
from jax.experimental.pallas import tpu as pltpu

_SR = 16000
_N_FFT = 512
_N_MELS = 80
_HOP = 160
_ALPHA, _DELTA, _R, _S, _EPS = 0.98, 2.0, 0.5, 0.025, 1e-6
_TC = 256
_UC = 4
_NB = _N_FFT // 2


def _mel_fbanks_np(n_freqs, f_min, f_max, n_mels, sr):
    all_freqs = np.linspace(0.0, sr / 2.0, n_freqs)

    def hz_to_mel(f):
        return 2595.0 * np.log10(1.0 + f / 700.0)

    def mel_to_hz(m):
        return 700.0 * (10.0 ** (m / 2595.0) - 1.0)

    m_pts = np.linspace(hz_to_mel(f_min), hz_to_mel(f_max), n_mels + 2)
    f_pts = mel_to_hz(m_pts)
    f_diff = f_pts[1:] - f_pts[:-1]
    slopes = f_pts[None, :] - all_freqs[:, None]
    down = -slopes[:, :-2] / f_diff[:-1]
    up = slopes[:, 2:] / f_diff[1:]
    return np.clip(np.minimum(down, up), 0.0, None)


def _consts():
    n = np.arange(_N_FFT, dtype=np.float64)
    win = 0.5 * (1.0 - np.cos(2.0 * np.pi * n / _N_FFT))
    k = np.arange(1, _NB + 1, dtype=np.float64)
    ang = 2.0 * np.pi * np.outer(n, k) / _N_FFT
    dft = np.concatenate([win[:, None] * np.cos(ang),
                          win[:, None] * np.sin(ang)], axis=1)
    fbt = _mel_fbanks_np(_N_FFT // 2 + 1, 0.0, _SR / 2.0,
                         _N_MELS, _SR)[1:_NB + 1].T
    t = np.arange(_TC, dtype=np.float64)
    lt = np.where(t[None, :] >= t[:, None],
                  _S * (1.0 - _S) ** (t[None, :] - t[:, None]), 0.0)
    dt = ((1.0 - _S) ** (t + 1.0))[None, :]
    dft32 = dft.astype(np.float32)
    dft_hi = dft32.astype(jnp.bfloat16)
    dft_lo = (dft32 - dft_hi.astype(np.float32)).astype(jnp.bfloat16)
    return (dft_hi, dft_lo, fbt.astype(np.float32),
            lt.astype(np.float32), dt.astype(np.float32))


_DFT_HI, _DFT_LO, _FBT, _LTM, _DTV = _consts()


def _split_dot(x, a_hi_ref, a_lo_ref, lo, hi):
    bf16, f32 = jnp.bfloat16, jnp.float32
    xh = x.astype(bf16)
    xl = (x - xh.astype(f32)).astype(bf16)
    ah = a_hi_ref[lo:hi, :]
    al = a_lo_ref[lo:hi, :]
    out = jnp.dot(xh, ah, preferred_element_type=f32)
    out = out + (jnp.dot(xh, al, preferred_element_type=f32)
                 + jnp.dot(xl, ah, preferred_element_type=f32))
    return out


def _body(x_ref, ah_ref, al_ref, fbt_ref, lt_ref, dt_ref, o_ref, s_ref):
    c = pl.program_id(1)
    f32 = jnp.float32
    bf16 = jnp.bfloat16
    vhs, vls = [], []
    for u in range(_UC):
        base = pl.multiple_of(c * (_UC * _TC) + u * _TC, 8)
        v = x_ref[0, pl.ds(base, _TC + 8), :]
        vh = v.astype(bf16)
        vhs.append(vh)
        vls.append((v - vh.astype(f32)).astype(bf16))
    accs = [None] * _UC
    for p, (lo, hi) in enumerate(((0, 160), (160, 320), (320, 480), (480, 512))):
        for u in range(_UC):
            if p < 3:
                sh = vhs[u][p:_TC + p]
                sl = vls[u][p:_TC + p]
            else:
                sh = vhs[u][3:_TC + 3, 0:32]
                sl = vls[u][3:_TC + 3, 0:32]
            d = jnp.dot(sh, ah_ref[lo:hi, :], preferred_element_type=f32)
            d = d + (jnp.dot(sh, al_ref[lo:hi, :], preferred_element_type=f32)
                     + jnp.dot(sl, ah_ref[lo:hi, :], preferred_element_type=f32))
            accs[u] = d if accs[u] is None else accs[u] + d
    xts = []
    for u in range(_UC):
        acc = accs[u]
        power = acc[:, :_NB] * acc[:, :_NB] + acc[:, _NB:] * acc[:, _NB:]
        xts.append(jax.lax.dot_general(fbt_ref[...], power, (((1,), (1,)), ((), ())),
                                       preferred_element_type=f32) + 1e-9)

    @pl.when(c == 0)
    def _():
        s_ref[...] = xts[0][:, 0:1]

    geom = [jnp.dot(x, lt_ref[...], preferred_element_type=f32) for x in xts]
    s_in = s_ref[...]
    for u in range(_UC):
        smooth = geom[u] + s_in * dt_ref[...]
        s_in = smooth[:, _TC - 1:_TC]
        uu = xts[u] * jnp.exp(-_ALPHA * jnp.log(smooth + _EPS)) + _DELTA
        o_ref[0, :, u * _TC:(u + 1) * _TC] = jnp.sqrt(uu) - _DELTA ** _R
    s_ref[...] = s_in


def kernel(waveform):
    b, s = waveform.shape
    t_frames = 1 + s // _HOP
    step = _UC * _TC
    nc = -(-t_frames // step)
    rows = nc * step + 8
    spad = rows * _HOP
    left = waveform[:, 256:0:-1]
    right = waveform[:, -2:-258:-1]
    z = jnp.zeros((b, spad - s - 2 * 256), waveform.dtype)
    xp = jnp.concatenate([left, waveform, right, z], axis=1).reshape(b, rows, _HOP)
    out = pl.pallas_call(
        _body,
        out_shape=jax.ShapeDtypeStruct((b, _N_MELS, t_frames), jnp.float32),
        grid=(b, nc),
        in_specs=[
            pl.BlockSpec((1, rows, _HOP), lambda bi, ci: (bi, 0, 0)),
            pl.BlockSpec((_N_FFT, 2 * _NB), lambda bi, ci: (0, 0)),
            pl.BlockSpec((_N_FFT, 2 * _NB), lambda bi, ci: (0, 0)),
            pl.BlockSpec((_N_MELS, _NB), lambda bi, ci: (0, 0)),
            pl.BlockSpec((_TC, _TC), lambda bi, ci: (0, 0)),
            pl.BlockSpec((1, _TC), lambda bi, ci: (0, 0)),
        ],
        out_specs=pl.BlockSpec((1, _N_MELS, _UC * _TC), lambda bi, ci: (bi, 0, ci)),
        scratch_shapes=[pltpu.VMEM((_N_MELS, 1), jnp.float32)],
        compiler_params=pltpu.CompilerParams(
            dimension_semantics=("parallel", "arbitrary"),
            vmem_limit_bytes=48 * 1024 * 1024,
        ),
        name="mel_pcen_fused",
    )(xp, jnp.asarray(_DFT_HI), jnp.asarray(_DFT_LO), jnp.asarray(_FBT),
      jnp.asarray(_LTM), jnp.asarray(_DTV))
    return out


# speedup vs baseline: 31.8128x; 1.3570x over previous
import numpy as np
import jax
import jax.numpy as jnp
from jax.experimental import pallas as pl
from jax.experimental.pallas import tpu as pltpu

_SR = 16000
_N_FFT = 512
_N_MELS = 80
_HOP = 160
_ALPHA, _DELTA, _R, _S, _EPS = 0.98, 2.0, 0.5, 0.025, 1e-6
_TC = 256
_UC = 4
_NB = _N_FFT // 2


def _mel_fbanks_np(n_freqs, f_min, f_max, n_mels, sr):
    all_freqs = np.linspace(0.0, sr / 2.0, n_freqs)

    def hz_to_mel(f):
        return 2595.0 * np.log10(1.0 + f / 700.0)

    def mel_to_hz(m):
        return 700.0 * (10.0 ** (m / 2595.0) - 1.0)

    m_pts = np.linspace(hz_to_mel(f_min), hz_to_mel(f_max), n_mels + 2)
    f_pts = mel_to_hz(m_pts)
    f_diff = f_pts[1:] - f_pts[:-1]
    slopes = f_pts[None, :] - all_freqs[:, None]
    down = -slopes[:, :-2] / f_diff[:-1]
    up = slopes[:, 2:] / f_diff[1:]
    return np.clip(np.minimum(down, up), 0.0, None)


def _consts():
    n = np.arange(_N_FFT, dtype=np.float64)
    win = 0.5 * (1.0 - np.cos(2.0 * np.pi * n / _N_FFT))
    k = np.arange(1, _NB + 1, dtype=np.float64)
    ang = 2.0 * np.pi * np.outer(n, k) / _N_FFT
    dft = np.concatenate([win[:, None] * np.cos(ang),
                          win[:, None] * np.sin(ang)], axis=1)
    fbt = _mel_fbanks_np(_N_FFT // 2 + 1, 0.0, _SR / 2.0,
                         _N_MELS, _SR)[1:_NB + 1].T
    t = np.arange(_TC, dtype=np.float64)
    lt = np.where(t[None, :] >= t[:, None],
                  _S * (1.0 - _S) ** (t[None, :] - t[:, None]), 0.0)
    dt = ((1.0 - _S) ** (t + 1.0))[None, :]
    dft32 = dft.astype(np.float32)
    dft_hi = dft32.astype(jnp.bfloat16)
    dft_lo = (dft32 - dft_hi.astype(np.float32)).astype(jnp.bfloat16)
    return (dft_hi, dft_lo, fbt.astype(np.float32),
            lt.astype(np.float32), dt.astype(np.float32))


_DFT_HI, _DFT_LO, _FBT, _LTM, _DTV = _consts()


def _split_dot(x, a_hi_ref, a_lo_ref, lo, hi):
    bf16, f32 = jnp.bfloat16, jnp.float32
    xh = x.astype(bf16)
    xl = (x - xh.astype(f32)).astype(bf16)
    ah = a_hi_ref[lo:hi, :]
    al = a_lo_ref[lo:hi, :]
    out = jnp.dot(xh, ah, preferred_element_type=f32)
    out = out + (jnp.dot(xh, al, preferred_element_type=f32)
                 + jnp.dot(xl, ah, preferred_element_type=f32))
    return out


def _body(xh_ref, ah_ref, fbt_ref, lt_ref, dt_ref, o_ref, s_ref):
    c = pl.program_id(1)
    f32 = jnp.float32
    vhs = []
    for u in range(_UC):
        base = pl.multiple_of(c * (_UC * _TC) + u * _TC, 8)
        vhs.append(xh_ref[0, pl.ds(base, _TC + 8), :])
    accs = [None] * _UC
    for u in range(_UC):
        for p, (lo, hi) in enumerate(((0, 160), (160, 320), (320, 480), (480, 512))):
            if p < 3:
                sh = vhs[u][p:_TC + p]
            else:
                sh = vhs[u][3:_TC + 3, 0:32]
            d = jnp.dot(sh, ah_ref[lo:hi, :], preferred_element_type=f32)
            accs[u] = d if accs[u] is None else accs[u] + d
    xts = []
    for u in range(_UC):
        acc = accs[u]
        power = acc[:, :_NB] * acc[:, :_NB] + acc[:, _NB:] * acc[:, _NB:]
        xts.append(jax.lax.dot_general(fbt_ref[...], power, (((1,), (1,)), ((), ())),
                                       preferred_element_type=f32) + 1e-9)

    @pl.when(c == 0)
    def _():
        s_ref[...] = xts[0][:, 0:1]

    geom = [jnp.dot(x, lt_ref[...], preferred_element_type=f32) for x in xts]
    s_in = s_ref[...]
    for u in range(_UC):
        smooth = geom[u] + s_in * dt_ref[...]
        s_in = smooth[:, _TC - 1:_TC]
        uu = xts[u] * jnp.exp(-_ALPHA * jnp.log(smooth + _EPS)) + _DELTA
        o_ref[0, :, u * _TC:(u + 1) * _TC] = jnp.sqrt(uu) - _DELTA ** _R
    s_ref[...] = s_in


def kernel(waveform):
    b, s = waveform.shape
    t_frames = 1 + s // _HOP
    step = _UC * _TC
    nc = -(-t_frames // step)
    rows = nc * step + 8
    spad = rows * _HOP
    left = waveform[:, 256:0:-1]
    right = waveform[:, -2:-258:-1]
    z = jnp.zeros((b, spad - s - 2 * 256), waveform.dtype)
    xp = jnp.concatenate([left, waveform, right, z], axis=1).reshape(b, rows, _HOP)
    xh = xp.astype(jnp.bfloat16)
    out = pl.pallas_call(
        _body,
        out_shape=jax.ShapeDtypeStruct((b, _N_MELS, t_frames), jnp.float32),
        grid=(b, nc),
        in_specs=[
            pl.BlockSpec((1, rows, _HOP), lambda bi, ci: (bi, 0, 0)),
            pl.BlockSpec((_N_FFT, 2 * _NB), lambda bi, ci: (0, 0)),
            pl.BlockSpec((_N_MELS, _NB), lambda bi, ci: (0, 0)),
            pl.BlockSpec((_TC, _TC), lambda bi, ci: (0, 0)),
            pl.BlockSpec((1, _TC), lambda bi, ci: (0, 0)),
        ],
        out_specs=pl.BlockSpec((1, _N_MELS, _UC * _TC), lambda bi, ci: (bi, 0, ci)),
        scratch_shapes=[pltpu.VMEM((_N_MELS, 1), jnp.float32)],
        compiler_params=pltpu.CompilerParams(
            dimension_semantics=("parallel", "arbitrary"),
            vmem_limit_bytes=48 * 1024 * 1024,
        ),
        name="mel_pcen_fused",
    )(xh, jnp.asarray(_DFT_HI), jnp.asarray(_FBT),
      jnp.asarray(_LTM), jnp.asarray(_DTV))
    return out


# speedup vs baseline: 38.9051x; 1.2229x over previous
import numpy as np
import jax
import jax.numpy as jnp
from jax.experimental import pallas as pl
from jax.experimental.pallas import tpu as pltpu

_SR = 16000
_N_FFT = 512
_N_MELS = 80
_HOP = 160
_ALPHA, _DELTA, _R, _S, _EPS = 0.98, 2.0, 0.5, 0.025, 1e-6
_TC = 256
_UC = 4
_NB = _N_FFT // 2
_VR = _TC + 16


def _mel_fbanks_np(n_freqs, f_min, f_max, n_mels, sr):
    all_freqs = np.linspace(0.0, sr / 2.0, n_freqs)

    def hz_to_mel(f):
        return 2595.0 * np.log10(1.0 + f / 700.0)

    def mel_to_hz(m):
        return 700.0 * (10.0 ** (m / 2595.0) - 1.0)

    m_pts = np.linspace(hz_to_mel(f_min), hz_to_mel(f_max), n_mels + 2)
    f_pts = mel_to_hz(m_pts)
    f_diff = f_pts[1:] - f_pts[:-1]
    slopes = f_pts[None, :] - all_freqs[:, None]
    down = -slopes[:, :-2] / f_diff[:-1]
    up = slopes[:, 2:] / f_diff[1:]
    return np.clip(np.minimum(down, up), 0.0, None)


def _consts():
    n = np.arange(_N_FFT, dtype=np.float64)
    win = 0.5 * (1.0 - np.cos(2.0 * np.pi * n / _N_FFT))
    k = np.arange(1, _NB + 1, dtype=np.float64)
    ang = 2.0 * np.pi * np.outer(n, k) / _N_FFT
    dft = np.concatenate([win[:, None] * np.cos(ang),
                          win[:, None] * np.sin(ang)], axis=1)
    ap = np.zeros((4 * _HOP, _N_FFT))
    ap[64:160] = dft[0:96]
    ap[160:320] = dft[96:256]
    ap[320:480] = dft[256:416]
    ap[480:576] = dft[416:512]
    fbt = _mel_fbanks_np(_N_FFT // 2 + 1, 0.0, _SR / 2.0,
                         _N_MELS, _SR)[1:_NB + 1].T
    t = np.arange(_TC, dtype=np.float64)
    lt = np.where(t[None, :] >= t[:, None],
                  _S * (1.0 - _S) ** (t[None, :] - t[:, None]), 0.0)
    dt = ((1.0 - _S) ** (t + 1.0))[None, :]
    return (ap.astype(jnp.bfloat16), fbt.astype(np.float32),
            lt.astype(np.float32), dt.astype(np.float32))


_APM, _FBT, _LTM, _DTV = _consts()


def _body(x_ref, e_ref, a_ref, fbt_ref, lt_ref, dt_ref, o_ref,
          s_ref, v0_ref, v3_ref):
    c = pl.program_id(1)
    nc = pl.num_programs(1)
    f32 = jnp.float32
    bf16 = jnp.bfloat16
    step = _UC * _TC
    rrows = x_ref.shape[1]

    @pl.when(c == 0)
    def _():
        v0_ref[0:8] = e_ref[0, 0:8, :]
        v0_ref[8:_VR] = x_ref[0, 0:_VR - 8, :]

    @pl.when(c != 0)
    def _():
        v0_ref[...] = x_ref[0, pl.ds(pl.multiple_of(c * step - 8, 8), _VR), :]

    vs = [v0_ref[...]]
    for u in range(1, _UC - 1):
        start = pl.multiple_of(c * step + u * _TC - 8, 8)
        vs.append(x_ref[0, pl.ds(start, _VR), :])

    n_avail = rrows - (nc * step - _TC - 8)

    @pl.when(c == nc - 1)
    def _():
        v3_ref[0:n_avail] = x_ref[0, rrows - n_avail:rrows, :]
        v3_ref[n_avail:n_avail + 8] = e_ref[0, 8:16, :]
        v3_ref[n_avail + 8:_VR] = jnp.zeros((_VR - n_avail - 8, _HOP), f32)

    @pl.when(c != nc - 1)
    def _():
        v3_ref[...] = x_ref[0, pl.ds(
            pl.multiple_of(c * step + (_UC - 1) * _TC - 8, 8), _VR), :]

    vs.append(v3_ref[...])

    accs = [None] * _UC
    for u in range(_UC):
        for p in range(4):
            sh = vs[u][6 + p:6 + p + _TC].astype(bf16)
            d = jnp.dot(sh, a_ref[p * _HOP:(p + 1) * _HOP, :],
                        preferred_element_type=f32)
            accs[u] = d if accs[u] is None else accs[u] + d
    xts = []
    for u in range(_UC):
        acc = accs[u]
        power = acc[:, :_NB] * acc[:, :_NB] + acc[:, _NB:] * acc[:, _NB:]
        xts.append(jax.lax.dot_general(fbt_ref[...], power, (((1,), (1,)), ((), ())),
                                       preferred_element_type=f32) + 1e-9)

    @pl.when(c == 0)
    def _():
        s_ref[...] = xts[0][:, 0:1]

    geom = [jnp.dot(x, lt_ref[...], preferred_element_type=f32) for x in xts]
    s_in = s_ref[...]
    for u in range(_UC):
        smooth = geom[u] + s_in * dt_ref[...]
        s_in = smooth[:, _TC - 1:_TC]
        uu = xts[u] * jnp.exp(-_ALPHA * jnp.log(smooth + _EPS)) + _DELTA
        o_ref[0, :, u * _TC:(u + 1) * _TC] = jnp.sqrt(uu) - _DELTA ** _R
    s_ref[...] = s_in


def kernel(waveform):
    b, s = waveform.shape
    t_frames = 1 + s // _HOP
    step = _UC * _TC
    nc = -(-t_frames // step)
    rrows = s // _HOP
    xr = waveform.reshape(b, rrows, _HOP)
    zpad = jnp.zeros((b, 1024), waveform.dtype)
    edges = jnp.concatenate(
        [zpad, waveform[:, 256:0:-1], waveform[:, -2:-258:-1], zpad],
        axis=1).reshape(b, 16, _HOP)
    out = pl.pallas_call(
        _body,
        out_shape=jax.ShapeDtypeStruct((b, _N_MELS, t_frames), jnp.float32),
        grid=(b, nc),
        in_specs=[
            pl.BlockSpec((1, rrows, _HOP), lambda bi, ci: (bi, 0, 0)),
            pl.BlockSpec((1, 16, _HOP), lambda bi, ci: (bi, 0, 0)),
            pl.BlockSpec((4 * _HOP, 2 * _NB), lambda bi, ci: (0, 0)),
            pl.BlockSpec((_N_MELS, _NB), lambda bi, ci: (0, 0)),
            pl.BlockSpec((_TC, _TC), lambda bi, ci: (0, 0)),
            pl.BlockSpec((1, _TC), lambda bi, ci: (0, 0)),
        ],
        out_specs=pl.BlockSpec((1, _N_MELS, _UC * _TC), lambda bi, ci: (bi, 0, ci)),
        scratch_shapes=[pltpu.VMEM((_N_MELS, 1), jnp.float32),
                        pltpu.VMEM((_VR, _HOP), jnp.float32),
                        pltpu.VMEM((_VR, _HOP), jnp.float32)],
        compiler_params=pltpu.CompilerParams(
            dimension_semantics=("parallel", "arbitrary"),
            vmem_limit_bytes=48 * 1024 * 1024,
        ),
        name="mel_pcen_fused",
    )(xr, edges, jnp.asarray(_APM), jnp.asarray(_FBT),
      jnp.asarray(_LTM), jnp.asarray(_DTV))
    return out
